# baseline (speedup 1.0000x reference)
import functools

import jax
import jax.numpy as jnp
from jax import lax
from jax.experimental import pallas as pl
from jax.experimental.pallas import tpu as pltpu
from jax.experimental.pallas import tpu_sc as plsc

VOCAB = 1000000
D = 50
DP = 56
BATCH = 4096
SEQ = 200

NUM_WORKERS = 32
EPW = BATCH // NUM_WORKERS
CHA = 128
CHB = SEQ - CHA


CONV_CHUNK = 500
N_CHUNKS = VOCAB // CONV_CHUNK


def _sc_pad56(flat50):
    mesh = plsc.VectorSubcoreMesh(core_axis_name="c", subcore_axis_name="s")
    per_worker = -(-N_CHUNKS // NUM_WORKERS)

    @functools.partial(
        pl.kernel,
        mesh=mesh,
        out_type=jax.ShapeDtypeStruct((VOCAB, DP), jnp.float32),
        compiler_params=pltpu.CompilerParams(use_tc_tiling_on_sc=False),
        scratch_types=[
            pltpu.VMEM((CONV_CHUNK * D,), jnp.float32),
            pltpu.VMEM((CONV_CHUNK, DP), jnp.float32),
        ],
    )
    def k(in_hbm, out_hbm, inv, outv):
        wid = lax.axis_index("s") * 2 + lax.axis_index("c")

        def chunk(i, carry):
            c = wid + i * NUM_WORKERS

            @pl.when(c < N_CHUNKS)
            def _():
                pltpu.sync_copy(
                    in_hbm.at[pl.ds(c * (CONV_CHUNK * D), CONV_CHUNK * D)], inv)

                def row(r, cc):
                    outv[r, pl.ds(0, 16)] = inv[pl.ds(r * D, 16)]
                    outv[r, pl.ds(16, 16)] = inv[pl.ds(r * D + 16, 16)]
                    outv[r, pl.ds(32, 16)] = inv[pl.ds(r * D + 32, 16)]
                    outv[r, pl.ds(34, 16)] = inv[pl.ds(r * D + 34, 16)]
                    return cc

                lax.fori_loop(0, CONV_CHUNK, row, 0)
                pltpu.sync_copy(
                    outv,
                    out_hbm.at[pl.ds(c * CONV_CHUNK, CONV_CHUNK)])

            return carry

        lax.fori_loop(0, per_worker, chunk, 0)

    return k(flat50)


def _sc_gather_pool(idx_flat, tab56):
    mesh = plsc.VectorSubcoreMesh(core_axis_name="c", subcore_axis_name="s")

    @functools.partial(
        pl.kernel,
        mesh=mesh,
        out_type=jax.ShapeDtypeStruct((BATCH * D,), jnp.float32),
        compiler_params=pltpu.CompilerParams(use_tc_tiling_on_sc=False),
        scratch_types=[
            pltpu.VMEM((EPW * SEQ,), jnp.int32),
            pltpu.VMEM((CHA, DP), jnp.float32),
            pltpu.VMEM((CHB, DP), jnp.float32),
            pltpu.VMEM((EPW * D,), jnp.float32),
            pltpu.SemaphoreType.DMA,
            pltpu.SemaphoreType.DMA,
        ],
    )
    def k(idx_hbm, tab_hbm, out_hbm, idx_v, bufa, bufb, outbuf, sem0, sem1):
        wid = lax.axis_index("s") * 2 + lax.axis_index("c")
        pltpu.sync_copy(idx_hbm.at[pl.ds(wid * (EPW * SEQ), EPW * SEQ)], idx_v)

        def accumulate(b):
            def body(r, accs):
                a0, a1, a2, a3 = accs
                a0 += bufa[r, pl.ds(0, 16)]
                a1 += bufa[r, pl.ds(16, 16)]
                a2 += bufa[r, pl.ds(32, 16)]
                a3 += bufa[r, pl.ds(34, 16)]
                return (a0, a1, a2, a3)

            def body2(r, accs):
                a0, a1, a2, a3 = accs
                a0 += bufb[r, pl.ds(0, 16)]
                a1 += bufb[r, pl.ds(16, 16)]
                a2 += bufb[r, pl.ds(32, 16)]
                a3 += bufb[r, pl.ds(34, 16)]
                return (a0, a1, a2, a3)

            z = jnp.zeros((16,), jnp.float32)
            accs = lax.fori_loop(0, CHA, body, (z, z, z, z))
            a0, a1, a2, a3 = lax.fori_loop(0, CHB, body2, accs)
            s = jnp.float32(1.0 / SEQ)
            off = b * D
            outbuf[pl.ds(off, 16)] = a0 * s
            outbuf[pl.ds(off + 16, 16)] = a1 * s
            outbuf[pl.ds(off + 32, 16)] = a2 * s
            outbuf[pl.ds(off + 34, 16)] = a3 * s

        def elem(b, carry):
            h0 = pltpu.async_copy(
                tab_hbm.at[idx_v.at[pl.ds(b * SEQ, CHA)]], bufa, sem0)
            h1 = pltpu.async_copy(
                tab_hbm.at[idx_v.at[pl.ds(b * SEQ + CHA, CHB)]], bufb, sem1)
            h0.wait()
            h1.wait()
            accumulate(b)
            return carry

        lax.fori_loop(0, EPW, elem, 0)
        pltpu.sync_copy(outbuf, out_hbm.at[pl.ds(wid * (EPW * D), EPW * D)])

    return k(idx_flat, tab56)


def _tc_mlp(pooled, W1, b1, gamma, beta, W2p, b2p):
    def body(x_ref, w1_ref, b1_ref, g_ref, be_ref, w2_ref, b2_ref, o_ref):
        x = x_ref[:]
        h = jnp.dot(x, w1_ref[:], preferred_element_type=jnp.float32) + b1_ref[:]
        mu = jnp.mean(h, axis=0, keepdims=True)
        d = h - mu
        var = jnp.mean(d * d, axis=0, keepdims=True)
        y = d * (g_ref[:] * lax.rsqrt(var + 1e-5)) + be_ref[:]
        o_ref[:] = jnp.dot(y, w2_ref[:], preferred_element_type=jnp.float32) + b2_ref[:]

    return pl.pallas_call(
        body,
        out_shape=jax.ShapeDtypeStruct((BATCH, 128), jnp.float32),
    )(pooled, W1, b1, gamma, beta, W2p, b2p)


def kernel(inp, table, W1, b1, gamma, beta, W2, b2):
    idx_flat = inp.astype(jnp.int32).reshape(BATCH * SEQ)
    tab56 = _sc_pad56(table.reshape(VOCAB * D))
    pooled = _sc_gather_pool(idx_flat, tab56).reshape(BATCH, D)
    W2p = jnp.pad(W2, ((0, 0), (0, 126)))
    b2p = jnp.pad(b2, (0, 126)).reshape(1, 128)
    out = _tc_mlp(
        pooled,
        W1,
        b1.reshape(1, 200),
        gamma.reshape(1, 200),
        beta.reshape(1, 200),
        W2p,
        b2p,
    )
    return out[:, :2]


# speedup vs baseline: 1.2611x; 1.2611x over previous
import functools

import jax
import jax.numpy as jnp
from jax import lax
from jax.experimental import pallas as pl
from jax.experimental.pallas import tpu as pltpu
from jax.experimental.pallas import tpu_sc as plsc

VOCAB = 1000000
D = 50
DP = 56
BATCH = 4096
SEQ = 200

NUM_WORKERS = 32
EPW = BATCH // NUM_WORKERS
CHA = 128
CHB = SEQ - CHA


def _sc_gather_pool(idx_flat, tab56):
    mesh = plsc.VectorSubcoreMesh(core_axis_name="c", subcore_axis_name="s")

    @functools.partial(
        pl.kernel,
        mesh=mesh,
        out_type=jax.ShapeDtypeStruct((BATCH * D,), jnp.float32),
        compiler_params=pltpu.CompilerParams(use_tc_tiling_on_sc=False),
        scratch_types=[
            pltpu.VMEM((EPW * SEQ,), jnp.int32),
            pltpu.VMEM((CHA, DP), jnp.float32),
            pltpu.VMEM((CHB, DP), jnp.float32),
            pltpu.VMEM((CHA, DP), jnp.float32),
            pltpu.VMEM((CHB, DP), jnp.float32),
            pltpu.VMEM((EPW * D,), jnp.float32),
            pltpu.SemaphoreType.DMA,
            pltpu.SemaphoreType.DMA,
        ],
    )
    def k(idx_hbm, tab_hbm, out_hbm, idx_v, bufa0, bufb0, bufa1, bufb1,
          outbuf, sem0, sem1):
        wid = lax.axis_index("s") * 2 + lax.axis_index("c")
        pltpu.sync_copy(idx_hbm.at[pl.ds(wid * (EPW * SEQ), EPW * SEQ)], idx_v)

        def accumulate(b, bufa, bufb):
            def body(r, accs):
                a0, a1, a2, a3 = accs
                a0 += bufa[r, pl.ds(0, 16)]
                a1 += bufa[r, pl.ds(16, 16)]
                a2 += bufa[r, pl.ds(32, 16)]
                a3 += bufa[r, pl.ds(34, 16)]
                return (a0, a1, a2, a3)

            def body2(r, accs):
                a0, a1, a2, a3 = accs
                a0 += bufb[r, pl.ds(0, 16)]
                a1 += bufb[r, pl.ds(16, 16)]
                a2 += bufb[r, pl.ds(32, 16)]
                a3 += bufb[r, pl.ds(34, 16)]
                return (a0, a1, a2, a3)

            z = jnp.zeros((16,), jnp.float32)
            accs = lax.fori_loop(0, CHA, body, (z, z, z, z), unroll=4)
            a0, a1, a2, a3 = lax.fori_loop(0, CHB, body2, accs, unroll=4)
            s = jnp.float32(1.0 / SEQ)
            off = b * D
            outbuf[pl.ds(off, 16)] = a0 * s
            outbuf[pl.ds(off + 16, 16)] = a1 * s
            outbuf[pl.ds(off + 32, 16)] = a2 * s
            outbuf[pl.ds(off + 34, 16)] = a3 * s

        def fire(b, bufa, bufb, sem):
            pltpu.async_copy(
                tab_hbm.at[idx_v.at[pl.ds(b * SEQ, CHA)]], bufa, sem)
            pltpu.async_copy(
                tab_hbm.at[idx_v.at[pl.ds(b * SEQ + CHA, CHB)]], bufb, sem)

        def wait_set(bufa, bufb, sem):
            pltpu.make_async_copy(
                tab_hbm.at[idx_v.at[pl.ds(0, CHA)]], bufa, sem).wait()
            pltpu.make_async_copy(
                tab_hbm.at[idx_v.at[pl.ds(0, CHB)]], bufb, sem).wait()

        fire(0, bufa0, bufb0, sem0)
        fire(1, bufa1, bufb1, sem1)

        def it(i, carry):
            b = 2 * i
            wait_set(bufa0, bufb0, sem0)
            accumulate(b, bufa0, bufb0)
            fire(b + 2, bufa0, bufb0, sem0)
            wait_set(bufa1, bufb1, sem1)
            accumulate(b + 1, bufa1, bufb1)
            fire(b + 3, bufa1, bufb1, sem1)
            return carry

        lax.fori_loop(0, EPW // 2 - 1, it, 0)
        wait_set(bufa0, bufb0, sem0)
        accumulate(EPW - 2, bufa0, bufb0)
        wait_set(bufa1, bufb1, sem1)
        accumulate(EPW - 1, bufa1, bufb1)
        pltpu.sync_copy(outbuf, out_hbm.at[pl.ds(wid * (EPW * D), EPW * D)])

    return k(idx_flat, tab56)


def _tc_mlp(pooled, W1, b1, gamma, beta, W2p, b2p):
    def body(x_ref, w1_ref, b1_ref, g_ref, be_ref, w2_ref, b2_ref, o_ref):
        x = x_ref[:]
        h = jnp.dot(x, w1_ref[:], preferred_element_type=jnp.float32) + b1_ref[:]
        mu = jnp.mean(h, axis=0, keepdims=True)
        d = h - mu
        var = jnp.mean(d * d, axis=0, keepdims=True)
        y = d * (g_ref[:] * lax.rsqrt(var + 1e-5)) + be_ref[:]
        o_ref[:] = jnp.dot(y, w2_ref[:], preferred_element_type=jnp.float32) + b2_ref[:]

    return pl.pallas_call(
        body,
        out_shape=jax.ShapeDtypeStruct((BATCH, 128), jnp.float32),
    )(pooled, W1, b1, gamma, beta, W2p, b2p)


def kernel(inp, table, W1, b1, gamma, beta, W2, b2):
    idx_flat = inp.astype(jnp.int32).reshape(BATCH * SEQ)
    tab56 = jnp.pad(table, ((0, 0), (0, DP - D)))
    pooled = _sc_gather_pool(idx_flat, tab56).reshape(BATCH, D)
    W2p = jnp.pad(W2, ((0, 0), (0, 126)))
    b2p = jnp.pad(b2, (0, 126)).reshape(1, 128)
    out = _tc_mlp(
        pooled,
        W1,
        b1.reshape(1, 200),
        gamma.reshape(1, 200),
        beta.reshape(1, 200),
        W2p,
        b2p,
    )
    return out[:, :2]


# speedup vs baseline: 1.6537x; 1.3114x over previous
import functools

import jax
import jax.numpy as jnp
from jax import lax
from jax.experimental import pallas as pl
from jax.experimental.pallas import tpu as pltpu
from jax.experimental.pallas import tpu_sc as plsc

VOCAB = 1000000
D = 50
DP = 128
BATCH = 4096
SEQ = 200

NUM_WORKERS = 32
EPW = BATCH // NUM_WORKERS
CHA = 128
CHB = SEQ - CHA


def _sc_gather_pool(idx_flat, tab56):
    mesh = plsc.VectorSubcoreMesh(core_axis_name="c", subcore_axis_name="s")

    @functools.partial(
        pl.kernel,
        mesh=mesh,
        out_type=jax.ShapeDtypeStruct((BATCH * D,), jnp.float32),
        compiler_params=pltpu.CompilerParams(use_tc_tiling_on_sc=False, needs_layout_passes=False),
        scratch_types=[
            pltpu.VMEM((EPW * SEQ,), jnp.int32),
            pltpu.VMEM((CHA, DP), jnp.float32),
            pltpu.VMEM((CHB, DP), jnp.float32),
            pltpu.VMEM((CHA, DP), jnp.float32),
            pltpu.VMEM((CHB, DP), jnp.float32),
            pltpu.VMEM((EPW * D,), jnp.float32),
            pltpu.SemaphoreType.DMA,
            pltpu.SemaphoreType.DMA,
        ],
    )
    def k(idx_hbm, tab_hbm, out_hbm, idx_v, bufa0, bufb0, bufa1, bufb1,
          outbuf, sem0, sem1):
        wid = lax.axis_index("s") * 2 + lax.axis_index("c")
        pltpu.sync_copy(idx_hbm.at[pl.ds(wid * (EPW * SEQ), EPW * SEQ)], idx_v)

        def accumulate(b, bufa, bufb):
            def body(r, accs):
                a0, a1, a2, a3 = accs
                a0 += bufa[r, pl.ds(0, 16)]
                a1 += bufa[r, pl.ds(16, 16)]
                a2 += bufa[r, pl.ds(32, 16)]
                a3 += bufa[r, pl.ds(34, 16)]
                return (a0, a1, a2, a3)

            def body2(r, accs):
                a0, a1, a2, a3 = accs
                a0 += bufb[r, pl.ds(0, 16)]
                a1 += bufb[r, pl.ds(16, 16)]
                a2 += bufb[r, pl.ds(32, 16)]
                a3 += bufb[r, pl.ds(34, 16)]
                return (a0, a1, a2, a3)

            z = jnp.zeros((16,), jnp.float32)
            accs = lax.fori_loop(0, CHA, body, (z, z, z, z), unroll=4)
            a0, a1, a2, a3 = lax.fori_loop(0, CHB, body2, accs, unroll=4)
            s = jnp.float32(1.0 / SEQ)
            off = b * D
            outbuf[pl.ds(off, 16)] = a0 * s
            outbuf[pl.ds(off + 16, 16)] = a1 * s
            outbuf[pl.ds(off + 32, 16)] = a2 * s
            outbuf[pl.ds(off + 34, 16)] = a3 * s

        def fire(b, bufa, bufb, sem):
            pltpu.async_copy(
                tab_hbm.at[idx_v.at[pl.ds(b * SEQ, CHA)]], bufa, sem)
            pltpu.async_copy(
                tab_hbm.at[idx_v.at[pl.ds(b * SEQ + CHA, CHB)]], bufb, sem)

        def wait_set(bufa, bufb, sem):
            pltpu.make_async_copy(
                tab_hbm.at[idx_v.at[pl.ds(0, CHA)]], bufa, sem).wait()
            pltpu.make_async_copy(
                tab_hbm.at[idx_v.at[pl.ds(0, CHB)]], bufb, sem).wait()

        fire(0, bufa0, bufb0, sem0)
        fire(1, bufa1, bufb1, sem1)

        def it(i, carry):
            b = 2 * i
            wait_set(bufa0, bufb0, sem0)
            accumulate(b, bufa0, bufb0)
            fire(b + 2, bufa0, bufb0, sem0)
            wait_set(bufa1, bufb1, sem1)
            accumulate(b + 1, bufa1, bufb1)
            fire(b + 3, bufa1, bufb1, sem1)
            return carry

        lax.fori_loop(0, EPW // 2 - 1, it, 0)
        wait_set(bufa0, bufb0, sem0)
        accumulate(EPW - 2, bufa0, bufb0)
        wait_set(bufa1, bufb1, sem1)
        accumulate(EPW - 1, bufa1, bufb1)
        pltpu.sync_copy(outbuf, out_hbm.at[pl.ds(wid * (EPW * D), EPW * D)])

    return k(idx_flat, tab56)


def _tc_mlp(pooled, W1, b1, gamma, beta, W2p, b2p):
    def body(x_ref, w1_ref, b1_ref, g_ref, be_ref, w2_ref, b2_ref, o_ref):
        x = x_ref[:]
        h = jnp.dot(x, w1_ref[:], preferred_element_type=jnp.float32) + b1_ref[:]
        mu = jnp.mean(h, axis=0, keepdims=True)
        d = h - mu
        var = jnp.mean(d * d, axis=0, keepdims=True)
        y = d * (g_ref[:] * lax.rsqrt(var + 1e-5)) + be_ref[:]
        o_ref[:] = jnp.dot(y, w2_ref[:], preferred_element_type=jnp.float32) + b2_ref[:]

    return pl.pallas_call(
        body,
        out_shape=jax.ShapeDtypeStruct((BATCH, 128), jnp.float32),
    )(pooled, W1, b1, gamma, beta, W2p, b2p)


def kernel(inp, table, W1, b1, gamma, beta, W2, b2):
    idx_flat = inp.astype(jnp.int32).reshape(BATCH * SEQ)
    tab56 = jnp.pad(table, ((0, 0), (0, DP - D)))
    pooled = _sc_gather_pool(idx_flat, tab56).reshape(BATCH, D)
    W2p = jnp.pad(W2, ((0, 0), (0, 126)))
    b2p = jnp.pad(b2, (0, 126)).reshape(1, 128)
    out = _tc_mlp(
        pooled,
        W1,
        b1.reshape(1, 200),
        gamma.reshape(1, 200),
        beta.reshape(1, 200),
        W2p,
        b2p,
    )
    return out[:, :2]


# speedup vs baseline: 1.6548x; 1.0006x over previous
import functools

import jax
import jax.numpy as jnp
from jax import lax
from jax.experimental import pallas as pl
from jax.experimental.pallas import tpu as pltpu
from jax.experimental.pallas import tpu_sc as plsc

VOCAB = 1000000
D = 50
DP = 128
BATCH = 4096
SEQ = 200

NUM_WORKERS = 32
EPW = BATCH // NUM_WORKERS
CHA = 128
CHB = SEQ - CHA


def _sc_gather_pool(idx_flat, tab56):
    mesh = plsc.VectorSubcoreMesh(core_axis_name="c", subcore_axis_name="s")

    @functools.partial(
        pl.kernel,
        mesh=mesh,
        out_type=jax.ShapeDtypeStruct((BATCH * D,), jnp.float32),
        compiler_params=pltpu.CompilerParams(needs_layout_passes=False),
        scratch_types=[
            pltpu.VMEM((EPW * SEQ,), jnp.int32),
            pltpu.VMEM((CHA, DP), jnp.float32),
            pltpu.VMEM((CHB, DP), jnp.float32),
            pltpu.VMEM((CHA, DP), jnp.float32),
            pltpu.VMEM((CHB, DP), jnp.float32),
            pltpu.VMEM((EPW * D,), jnp.float32),
            pltpu.SemaphoreType.DMA,
            pltpu.SemaphoreType.DMA,
        ],
    )
    def k(idx_hbm, tab_hbm, out_hbm, idx_v, bufa0, bufb0, bufa1, bufb1,
          outbuf, sem0, sem1):
        wid = lax.axis_index("s") * 2 + lax.axis_index("c")
        pltpu.sync_copy(idx_hbm.at[pl.ds(wid * (EPW * SEQ), EPW * SEQ)], idx_v)

        def accumulate(b, bufa, bufb):
            def body(r, accs):
                a0, a1, a2, a3 = accs
                a0 += bufa[r, pl.ds(0, 16)]
                a1 += bufa[r, pl.ds(16, 16)]
                a2 += bufa[r, pl.ds(32, 16)]
                a3 += bufa[r, pl.ds(34, 16)]
                return (a0, a1, a2, a3)

            def body2(r, accs):
                a0, a1, a2, a3 = accs
                a0 += bufb[r, pl.ds(0, 16)]
                a1 += bufb[r, pl.ds(16, 16)]
                a2 += bufb[r, pl.ds(32, 16)]
                a3 += bufb[r, pl.ds(34, 16)]
                return (a0, a1, a2, a3)

            z = jnp.zeros((16,), jnp.float32)
            accs = lax.fori_loop(0, CHA, body, (z, z, z, z), unroll=4)
            a0, a1, a2, a3 = lax.fori_loop(0, CHB, body2, accs, unroll=4)
            s = jnp.float32(1.0 / SEQ)
            off = b * D
            outbuf[pl.ds(off, 16)] = a0 * s
            outbuf[pl.ds(off + 16, 16)] = a1 * s
            outbuf[pl.ds(off + 32, 16)] = a2 * s
            outbuf[pl.ds(off + 34, 16)] = a3 * s

        def fire(b, bufa, bufb, sem):
            pltpu.async_copy(
                tab_hbm.at[idx_v.at[pl.ds(b * SEQ, CHA)]], bufa, sem)
            pltpu.async_copy(
                tab_hbm.at[idx_v.at[pl.ds(b * SEQ + CHA, CHB)]], bufb, sem)

        def wait_set(bufa, bufb, sem):
            pltpu.make_async_copy(
                tab_hbm.at[idx_v.at[pl.ds(0, CHA)]], bufa, sem).wait()
            pltpu.make_async_copy(
                tab_hbm.at[idx_v.at[pl.ds(0, CHB)]], bufb, sem).wait()

        fire(0, bufa0, bufb0, sem0)
        fire(1, bufa1, bufb1, sem1)

        def it(i, carry):
            b = 2 * i
            wait_set(bufa0, bufb0, sem0)
            accumulate(b, bufa0, bufb0)
            fire(b + 2, bufa0, bufb0, sem0)
            wait_set(bufa1, bufb1, sem1)
            accumulate(b + 1, bufa1, bufb1)
            fire(b + 3, bufa1, bufb1, sem1)
            return carry

        lax.fori_loop(0, EPW // 2 - 1, it, 0)
        wait_set(bufa0, bufb0, sem0)
        accumulate(EPW - 2, bufa0, bufb0)
        wait_set(bufa1, bufb1, sem1)
        accumulate(EPW - 1, bufa1, bufb1)
        pltpu.sync_copy(outbuf, out_hbm.at[pl.ds(wid * (EPW * D), EPW * D)])

    return k(idx_flat, tab56)


def _tc_mlp(pooled, W1, b1, gamma, beta, W2p, b2p):
    def body(x_ref, w1_ref, b1_ref, g_ref, be_ref, w2_ref, b2_ref, o_ref):
        x = x_ref[:]
        h = jnp.dot(x, w1_ref[:], preferred_element_type=jnp.float32) + b1_ref[:]
        mu = jnp.mean(h, axis=0, keepdims=True)
        d = h - mu
        var = jnp.mean(d * d, axis=0, keepdims=True)
        y = d * (g_ref[:] * lax.rsqrt(var + 1e-5)) + be_ref[:]
        o_ref[:] = jnp.dot(y, w2_ref[:], preferred_element_type=jnp.float32) + b2_ref[:]

    return pl.pallas_call(
        body,
        out_shape=jax.ShapeDtypeStruct((BATCH, 128), jnp.float32),
    )(pooled, W1, b1, gamma, beta, W2p, b2p)


def kernel(inp, table, W1, b1, gamma, beta, W2, b2):
    idx_flat = inp.astype(jnp.int32).reshape(BATCH * SEQ)
    tab56 = jnp.pad(table, ((0, 0), (0, DP - D)))
    pooled = _sc_gather_pool(idx_flat, tab56).reshape(BATCH, D)
    W2p = jnp.pad(W2, ((0, 0), (0, 126)))
    b2p = jnp.pad(b2, (0, 126)).reshape(1, 128)
    out = _tc_mlp(
        pooled,
        W1,
        b1.reshape(1, 200),
        gamma.reshape(1, 200),
        beta.reshape(1, 200),
        W2p,
        b2p,
    )
    return out[:, :2]
